# baseline (speedup 1.0000x reference)
import jax
import jax.numpy as jnp
from jax.experimental import pallas as pl
from jax.experimental.pallas import tpu as pltpu

_P = 40
_V = 21
_H1 = 20
_H2 = 5
_FLAT = _P * _V


def _fused_kernel(x_ref, wfT_ref, w2T_ref, pk_ref, o_ref):
    idxT = x_ref[...].astype(jnp.int32)

    one = jnp.float32(1.0)
    zero = jnp.float32(0.0)
    oh = jnp.concatenate(
        [jnp.where(idxT == v, one, zero) for v in range(_V)], axis=0)

    pk = pk_ref[...]
    b1c = pk[:, 0:1]
    b2c = pk[:_H2, 1:2]
    w3c = pk[:_H2, 2:3]
    b3c = pk[0:1, 3:4]

    h1 = jnp.dot(wfT_ref[...], oh,
                 preferred_element_type=jnp.float32) + b1c
    h1 = jnp.maximum(h1, 0.0)

    h2 = jnp.dot(w2T_ref[...], h1,
                 preferred_element_type=jnp.float32) + b2c
    h2 = jnp.maximum(h2, 0.0)

    o_ref[...] = jnp.sum(h2 * w3c, axis=0, keepdims=True) + b3c


def kernel(x_idx, wfT, w2T, packed):
    B = x_idx.shape[0]
    TB = 32768
    xT = x_idx.T.astype(jnp.int8)
    grid = pl.cdiv(B, TB)
    out = pl.pallas_call(
        _fused_kernel,
        out_shape=jax.ShapeDtypeStruct((1, B), jnp.float32),
        grid=(grid,),
        in_specs=[
            pl.BlockSpec((_P, TB), lambda i: (0, i)),
            pl.BlockSpec((_H1, _FLAT), lambda i: (0, 0)),
            pl.BlockSpec((_H2, _H1), lambda i: (0, 0)),
            pl.BlockSpec((_H1, 4), lambda i: (0, 0)),
        ],
        out_specs=pl.BlockSpec((1, TB), lambda i: (0, i)),
        compiler_params=pltpu.CompilerParams(
            dimension_semantics=("parallel",),
            vmem_limit_bytes=100 << 20),
    )(xT, wfT, w2T, packed)
    return out.reshape(B, 1)


# speedup vs baseline: 1.2344x; 1.2344x over previous
import jax
import jax.numpy as jnp
from jax.experimental import pallas as pl
from jax.experimental.pallas import tpu as pltpu

_P = 40
_V = 21
_H1 = 20
_H2 = 5
_FLAT = _P * _V


def _fused_kernel(x_ref, wfT_ref, w2T_ref, pk_ref, o_ref):
    idxT = x_ref[...]

    one = jnp.float32(1.0)
    zero = jnp.float32(0.0)
    oh = jnp.concatenate(
        [jnp.where(idxT == v, one, zero) for v in range(_V)], axis=0)

    pk = pk_ref[...]
    b1c = pk[:, 0:1]
    b2c = pk[:_H2, 1:2]
    w3c = pk[:_H2, 2:3]
    b3c = pk[0:1, 3:4]

    h1 = jnp.dot(wfT_ref[...], oh,
                 preferred_element_type=jnp.float32) + b1c
    h1 = jnp.maximum(h1, 0.0)

    h2 = jnp.dot(w2T_ref[...], h1,
                 preferred_element_type=jnp.float32) + b2c
    h2 = jnp.maximum(h2, 0.0)

    o_ref[...] = jnp.sum(h2 * w3c, axis=0, keepdims=True) + b3c


def kernel(x_idx, wfT, w2T, packed):
    B = x_idx.shape[0]
    TB = 32768
    xT = x_idx.T
    grid = pl.cdiv(B, TB)
    out = pl.pallas_call(
        _fused_kernel,
        out_shape=jax.ShapeDtypeStruct((1, B), jnp.float32),
        grid=(grid,),
        in_specs=[
            pl.BlockSpec((_P, TB), lambda i: (0, i)),
            pl.BlockSpec((_H1, _FLAT), lambda i: (0, 0)),
            pl.BlockSpec((_H2, _H1), lambda i: (0, 0)),
            pl.BlockSpec((_H1, 4), lambda i: (0, 0)),
        ],
        out_specs=pl.BlockSpec((1, TB), lambda i: (0, i)),
        compiler_params=pltpu.CompilerParams(
            dimension_semantics=("parallel",),
            vmem_limit_bytes=100 << 20),
    )(xT, wfT, w2T, packed)
    return out.reshape(B, 1)
